# Initial kernel scaffold; baseline (speedup 1.0000x reference)
import jax
import jax.numpy as jnp
from jax.experimental import pallas as pl


def kernel(x, W, b):
    raise NotImplementedError("write your pallas kernel here")


# speedup vs baseline: 2.0328x; 2.0328x over previous
import numpy as np
import jax
import jax.numpy as jnp
from jax.experimental import pallas as pl
from jax.experimental.pallas import tpu as pltpu

_B, _T, _D = 4, 8192, 768
_K = 819
_EMPH = 1.5
_NR, _NC = 64, 128


def _attn_body(x_ref, w_ref, b_ref, o_ref, e_ref):
    wrow = w_ref[...].astype(jnp.bfloat16).astype(jnp.float32)

    def p1(r, carry):
        xr = x_ref[r].astype(jnp.bfloat16).astype(jnp.float32)
        z = jnp.sum(xr * wrow, axis=-1)
        e_ref[pl.ds(r, 1), :] = z.reshape(1, _NC)
        return carry

    jax.lax.fori_loop(0, _NR, p1, 0, unroll=2)

    e2 = jnp.tanh(e_ref[...] + b_ref[0, 0])
    m = jnp.max(e2)
    p2 = jnp.exp(e2 - m)
    zsum = jnp.sum(p2)
    bits = jax.lax.bitcast_convert_type(p2, jnp.int32)
    tau = jnp.int32(-(2 ** 31))
    for bit in range(31, -1, -1):
        c = jnp.int32(np.uint32(1 << bit).astype(np.int32))
        cand = tau ^ c
        cnt = jnp.sum(jnp.where(bits >= cand, 1, 0))
        tau = jnp.where(cnt >= _K, cand, tau)
    wgt = p2 * jnp.where(bits >= tau, _EMPH / zsum, 1.0 / zsum)
    e_ref[...] = wgt

    def p3(r, acc):
        xr = x_ref[r]
        wc = e_ref[r, :].reshape(_NC, 1)
        prod = xr * wc
        return acc + jnp.sum(prod.reshape(16, 8, _D), axis=0)

    acc = jax.lax.fori_loop(0, _NR, p3, jnp.zeros((8, _D), jnp.float32),
                            unroll=2)
    o_ref[...] = jnp.sum(acc, axis=0, keepdims=True)


def kernel(x, W, b):
    x4 = x.reshape(_B, _NR, _NC, _D)
    w2 = W.reshape(1, _D)
    b2 = b.reshape(1, 1)
    out = pl.pallas_call(
        _attn_body,
        grid=(_B,),
        in_specs=[
            pl.BlockSpec((None, _NR, _NC, _D), lambda i: (i, 0, 0, 0)),
            pl.BlockSpec((1, _D), lambda i: (0, 0)),
            pl.BlockSpec((1, 1), lambda i: (0, 0)),
        ],
        out_specs=pl.BlockSpec((None, 1, _D), lambda i: (i, 0, 0)),
        out_shape=jax.ShapeDtypeStruct((_B, 1, _D), jnp.float32),
        scratch_shapes=[pltpu.VMEM((_NR, _NC), jnp.float32)],
        compiler_params=pltpu.CompilerParams(
            dimension_semantics=("arbitrary",),
        ),
    )(x4, w2, b2)
    return out
